# Initial kernel scaffold; baseline (speedup 1.0000x reference)
import jax
import jax.numpy as jnp
from jax.experimental import pallas as pl


def kernel(obs, wblob):
    raise NotImplementedError("write your pallas kernel here")


# speedup vs baseline: 1.5422x; 1.5422x over previous
import functools

import jax
import jax.numpy as jnp
from jax import lax
from jax.experimental import pallas as pl
from jax.experimental.pallas import tpu as pltpu

_N_INPUT = 8
_HIDDEN = 32
_N_OUTPUT = 3
_N_ATOMS = 16
_OA = _N_OUTPUT * _N_ATOMS


def _fused_kernel(x_ref, w_ref, o_ref, *, tb, chunk):
    w0 = w_ref[0]
    w1 = w_ref[1]
    w2 = w_ref[2]
    W1 = w0[:_N_INPUT, :_HIDDEN]
    b1 = w0[_N_INPUT:_N_INPUT + 1, :_HIDDEN]
    W2 = w1[:_HIDDEN, :_HIDDEN]
    b2 = w1[_HIDDEN:_HIDDEN + 1, :_HIDDEN]
    W3 = w2[:_HIDDEN, :_OA]
    b3 = w2[_HIDDEN:_HIDDEN + 1, :_OA]
    rg = lax.broadcasted_iota(jnp.int32, (_OA, _OA), 0) // _N_ATOMS
    cg = lax.broadcasted_iota(jnp.int32, (_OA, _OA), 1) // _N_ATOMS
    G = (rg == cg).astype(jnp.float32)

    for j in range(tb // chunk):
        x = x_ref[pl.ds(j * chunk, chunk), :]
        h1 = jnp.maximum(
            jnp.dot(x, W1, preferred_element_type=jnp.float32) + b1, 0.0)
        h2 = jnp.maximum(
            jnp.dot(h1, W2, preferred_element_type=jnp.float32) + b2, 0.0)
        lg = jnp.dot(h2, W3, preferred_element_type=jnp.float32) + b3
        m = jnp.max(lg, axis=-1, keepdims=True)
        e = jnp.exp(lg - m)
        gsum = jnp.dot(e, G, preferred_element_type=jnp.float32)
        p = e * pl.reciprocal(gsum, approx=True)
        base = 3 * j * chunk
        for g in range(_N_OUTPUT):
            o_ref[pl.ds(base + g, chunk, 3), :] = \
                p[:, g * _N_ATOMS:(g + 1) * _N_ATOMS]


def kernel(obs, wblob):
    obs = jnp.asarray(obs, jnp.float32)
    if obs.ndim == 1:
        obs = obs[None, :]
    B = obs.shape[0]
    TB = 2048
    if B % TB != 0:
        TB = B
    CHUNK = 256 if TB % 256 == 0 else TB
    P = wblob.shape[-1]

    body = functools.partial(_fused_kernel, tb=TB, chunk=CHUNK)
    out = pl.pallas_call(
        body,
        out_shape=jax.ShapeDtypeStruct((B * _N_OUTPUT, _N_ATOMS), jnp.float32),
        grid=(B // TB,),
        in_specs=[
            pl.BlockSpec((TB, _N_INPUT), lambda i: (i, 0)),
            pl.BlockSpec((3, P, P), lambda i: (0, 0, 0)),
        ],
        out_specs=pl.BlockSpec((TB * _N_OUTPUT, _N_ATOMS), lambda i: (i, 0)),
        compiler_params=pltpu.CompilerParams(
            dimension_semantics=("parallel",)),
    )(obs, wblob)

    return out.reshape(1, B * _N_OUTPUT, _N_ATOMS)
